# baseline (speedup 1.0000x reference)
import functools

import jax
import jax.numpy as jnp
import numpy as np
from jax import lax
from jax.experimental import pallas as pl
from jax.experimental.pallas import tpu as pltpu
from jax.experimental.pallas import tpu_sc as plsc

_DIM_USED = np.array(
    [6, 7, 8, 9, 10, 11, 12, 13, 14, 15, 16, 17, 21, 22, 23, 24, 25, 26,
     27, 28, 29, 30, 31, 32, 36, 37, 38, 39, 40, 41, 42, 43, 44, 45, 46,
     47, 51, 52, 53, 54, 55, 56, 57, 58, 59, 63, 64, 65, 66, 67, 68, 75,
     76, 77, 78, 79, 80, 81, 82, 83, 87, 88, 89, 90, 91, 92],
    dtype=np.int32)

_B, _T, _C = 4096, 50, 96
_K = _DIM_USED.shape[0]
_ROWS = _B * _T
_R = 320
_UNROLL = 8


@functools.lru_cache(maxsize=1)
def _build():
    info = plsc.get_sparse_core_info()
    nc, ns, lanes = info.num_cores, info.num_subcores, info.num_lanes
    nw = nc * ns
    rows_w = _ROWS // nw
    n_chunks = rows_w // _R
    ng = _R * _K // lanes

    idx_np = (np.arange(_R)[:, None] * _C + _DIM_USED[None, :]).astype(
        np.int32).reshape(-1)

    mesh = plsc.VectorSubcoreMesh(core_axis_name="c", subcore_axis_name="s")

    @functools.partial(
        pl.kernel,
        mesh=mesh,
        compiler_params=pltpu.CompilerParams(needs_layout_passes=False),
        out_type=jax.ShapeDtypeStruct((_ROWS * _K,), jnp.float32),
        scratch_types=[
            pltpu.VMEM((_R * _K,), jnp.int32),
            pltpu.VMEM((_R * _C,), jnp.float32),
            pltpu.VMEM((_R * _K,), jnp.float32),
        ],
    )
    def sc_select(x_hbm, idx_hbm, out_hbm, idx_v, in_v, out_v):
        wid = lax.axis_index("s") * nc + lax.axis_index("c")
        pltpu.sync_copy(idx_hbm, idx_v)
        row0 = wid * rows_w

        def chunk_body(c, carry):
            base = row0 + c * _R
            src_off = pl.multiple_of(base * _C, 8)
            dst_off = pl.multiple_of(base * _K, 8)
            pltpu.sync_copy(x_hbm.at[pl.ds(src_off, _R * _C)], in_v)

            def grp(g, carry2):
                for u in range(_UNROLL):
                    o = (g * _UNROLL + u) * lanes
                    iv = idx_v[pl.ds(o, lanes)]
                    out_v[pl.ds(o, lanes)] = plsc.load_gather(in_v, [iv])
                return carry2

            lax.fori_loop(0, ng // _UNROLL, grp, 0)
            pltpu.sync_copy(out_v, out_hbm.at[pl.ds(dst_off, _R * _K)])
            return carry

        lax.fori_loop(0, n_chunks, chunk_body, 0)

    return sc_select


def kernel(observed_pose):
    sc_select = _build()
    x = observed_pose.reshape(_ROWS * _C)
    idx = jnp.asarray(
        (np.arange(_R)[:, None] * _C + _DIM_USED[None, :]).astype(
            np.int32).reshape(-1))
    out = sc_select(x, idx)
    return out.reshape(_B, _T, _K)
